# Initial kernel scaffold; baseline (speedup 1.0000x reference)
import jax
import jax.numpy as jnp
from jax.experimental import pallas as pl


def kernel(X, edge_index, W, b):
    raise NotImplementedError("write your pallas kernel here")


# speedup vs baseline: 20.4072x; 20.4072x over previous
import functools

import jax
import jax.numpy as jnp
from jax import lax
from jax.experimental import pallas as pl
from jax.experimental.pallas import tpu as pltpu
from jax.experimental.pallas import tpu_sc as plsc

NC = 2
NS = 16
NW = NC * NS
L = 16
K = 128


def _sc_mesh():
  return plsc.VectorSubcoreMesh(core_axis_name="c", subcore_axis_name="s")


def kernel(X, edge_index, W, b):
  N, Din = X.shape
  Dout = W.shape[1]
  E = edge_index.shape[1]

  row_unit = NS * K
  NPAD = ((N + 1 + row_unit - 1) // row_unit) * row_unit
  TPR = NPAD // NS

  EPW = ((E + NW * K - 1) // (NW * K)) * K
  C = EPW // K
  EPAD = EPW * NW

  src = edge_index[0].astype(jnp.int32)
  dst = edge_index[1].astype(jnp.int32)
  pad = jnp.full((EPAD - E,), N, dtype=jnp.int32)
  srcp = jnp.concatenate([src, pad]).reshape(NW, C, K)
  dstp = jnp.concatenate([dst, pad]).reshape(NW, C, K)
  Xp = jnp.pad(X, ((0, NPAD - N), (0, 0)))

  @functools.partial(
      pl.kernel,
      out_type=jax.ShapeDtypeStruct((NC, NPAD), jnp.float32),
      mesh=_sc_mesh(),
      scratch_types=[
          pltpu.VMEM((C, K), jnp.int32),
          pltpu.VMEM((K,), jnp.float32),
          pltpu.VMEM((TPR,), jnp.float32),
          pltpu.VMEM_SHARED((NPAD,), jnp.float32),
      ],
  )
  def deg_kernel(dst_hbm, out_hbm, idx_v, ones_v, zero_v, deg_sp):
    c = lax.axis_index("c")
    s = lax.axis_index("s")
    wid = c * NS + s
    pltpu.sync_copy(dst_hbm.at[wid], idx_v)
    for i in range(K // L):
      ones_v[pl.ds(i * L, L)] = jnp.ones((L,), jnp.float32)
    for i in range(TPR // L):
      zero_v[pl.ds(i * L, L)] = jnp.zeros((L,), jnp.float32)
    pltpu.sync_copy(zero_v, deg_sp.at[pl.ds(s * TPR, TPR)])
    plsc.subcore_barrier()

    def body(j, carry):
      pltpu.sync_copy(ones_v, deg_sp.at[idx_v.at[j]], add=True)
      return carry

    lax.fori_loop(0, C, body, 0)
    plsc.subcore_barrier()
    pltpu.sync_copy(deg_sp.at[pl.ds(s * TPR, TPR)],
                    out_hbm.at[c, pl.ds(s * TPR, TPR)])

  degp = deg_kernel(dstp)
  deg_t = degp.T

  MB = 512

  def mm_body(x_ref, w_ref, b_ref, dg_ref, g_ref):
    d = dg_ref[:, 0:1] + dg_ref[:, 1:2] + 1.0
    dinv = lax.rsqrt(jnp.maximum(d, 1.0))
    h = jnp.dot(x_ref[...], w_ref[...],
                preferred_element_type=jnp.float32) + b_ref[...]
    g_ref[...] = h * dinv

  g = pl.pallas_call(
      mm_body,
      grid=(NPAD // MB,),
      in_specs=[
          pl.BlockSpec((MB, Din), lambda i: (i, 0)),
          pl.BlockSpec((Din, Dout), lambda i: (0, 0)),
          pl.BlockSpec((1, Dout), lambda i: (0, 0)),
          pl.BlockSpec((MB, NC), lambda i: (i, 0)),
      ],
      out_specs=pl.BlockSpec((MB, Dout), lambda i: (i, 0)),
      out_shape=jax.ShapeDtypeStruct((NPAD, Dout), jnp.float32),
  )(Xp, W, b.reshape(1, Dout), deg_t)

  @functools.partial(
      pl.kernel,
      out_type=jax.ShapeDtypeStruct((NC, NPAD, Dout), jnp.float32),
      mesh=_sc_mesh(),
      scratch_types=[
          pltpu.VMEM((C, K), jnp.int32),
          pltpu.VMEM((C, K), jnp.int32),
          pltpu.VMEM((K, Dout), jnp.float32),
          pltpu.VMEM_SHARED((NPAD, Dout), jnp.float32),
          pltpu.SemaphoreType.DMA,
      ],
  )
  def scat_kernel(g_hbm, src_hbm, dst_hbm, out_hbm,
                  si_v, di_v, rows_v, acc_sp, sem):
    c = lax.axis_index("c")
    s = lax.axis_index("s")
    wid = c * NS + s
    pltpu.sync_copy(src_hbm.at[wid], si_v)
    pltpu.sync_copy(dst_hbm.at[wid], di_v)

    def zbody(i, carry):
      for jj in range(Dout // L):
        rows_v[i, pl.ds(jj * L, L)] = jnp.zeros((L,), jnp.float32)
      return carry

    lax.fori_loop(0, K, zbody, 0)
    for r in range(TPR // K):
      pltpu.sync_copy(rows_v, acc_sp.at[pl.ds(s * TPR + r * K, K)])
    plsc.subcore_barrier()

    def body(j, carry):
      pltpu.async_copy(g_hbm.at[si_v.at[j]], rows_v, sem).wait()
      pltpu.sync_copy(rows_v, acc_sp.at[di_v.at[j]], add=True)
      return carry

    lax.fori_loop(0, C, body, 0)
    plsc.subcore_barrier()
    for r in range(TPR // K):
      pltpu.sync_copy(acc_sp.at[pl.ds(s * TPR + r * K, K)],
                      out_hbm.at[c, pl.ds(s * TPR + r * K, K)])

  accp = scat_kernel(g, srcp, dstp)

  MB2 = 1000

  def fin_body(a0_ref, a1_ref, g_ref, dg_ref, o_ref):
    d = dg_ref[:, 0:1] + dg_ref[:, 1:2] + 1.0
    dinv = lax.rsqrt(jnp.maximum(d, 1.0))
    tot = a0_ref[...] + a1_ref[...] + g_ref[...]
    o_ref[...] = jnp.maximum(tot * dinv, 0.0)

  out = pl.pallas_call(
      fin_body,
      grid=(N // MB2,),
      in_specs=[
          pl.BlockSpec((MB2, Dout), lambda i: (i, 0)),
          pl.BlockSpec((MB2, Dout), lambda i: (i, 0)),
          pl.BlockSpec((MB2, Dout), lambda i: (i, 0)),
          pl.BlockSpec((MB2, NC), lambda i: (i, 0)),
      ],
      out_specs=pl.BlockSpec((MB2, Dout), lambda i: (i, 0)),
      out_shape=jax.ShapeDtypeStruct((N, Dout), jnp.float32),
  )(accp[0], accp[1], g, deg_t)

  return out


# speedup vs baseline: 30.3367x; 1.4866x over previous
import functools

import jax
import jax.numpy as jnp
from jax import lax
from jax.experimental import pallas as pl
from jax.experimental.pallas import tpu as pltpu
from jax.experimental.pallas import tpu_sc as plsc

NC = 2
NS = 16
NW = NC * NS
L = 16
K = 128


def _sc_mesh():
  return plsc.VectorSubcoreMesh(core_axis_name="c", subcore_axis_name="s")


def kernel(X, edge_index, W, b):
  N, Din = X.shape
  Dout = W.shape[1]
  E = edge_index.shape[1]

  row_unit = NS * K
  NPAD = ((N + 1 + row_unit - 1) // row_unit) * row_unit
  TPR = NPAD // NS

  EPW = ((E + NW * K - 1) // (NW * K)) * K
  C = EPW // K
  EPAD = EPW * NW

  src = edge_index[0].astype(jnp.int32)
  dst = edge_index[1].astype(jnp.int32)
  pad = N + jnp.arange(EPAD - E, dtype=jnp.int32) % (NPAD - N)
  srcp = jnp.concatenate([src, pad]).reshape(NW, C, K)
  dstp = jnp.concatenate([dst, pad]).reshape(NW, C, K)
  Xp = jnp.pad(X, ((0, NPAD - N), (0, 0)))

  @functools.partial(
      pl.kernel,
      out_type=jax.ShapeDtypeStruct((NC, NPAD), jnp.float32),
      mesh=_sc_mesh(),
      scratch_types=[
          pltpu.VMEM((C, K), jnp.int32),
          pltpu.VMEM((K,), jnp.float32),
          pltpu.VMEM((TPR,), jnp.float32),
          pltpu.VMEM_SHARED((NPAD,), jnp.float32),
      ],
  )
  def deg_kernel(dst_hbm, out_hbm, idx_v, ones_v, zero_v, deg_sp):
    c = lax.axis_index("c")
    s = lax.axis_index("s")
    wid = c * NS + s
    pltpu.sync_copy(dst_hbm.at[wid], idx_v)
    for i in range(K // L):
      ones_v[pl.ds(i * L, L)] = jnp.ones((L,), jnp.float32)
    for i in range(TPR // L):
      zero_v[pl.ds(i * L, L)] = jnp.zeros((L,), jnp.float32)
    pltpu.sync_copy(zero_v, deg_sp.at[pl.ds(s * TPR, TPR)])
    plsc.subcore_barrier()

    def body(j, carry):
      pltpu.sync_copy(ones_v, deg_sp.at[idx_v.at[j]], add=True)
      return carry

    lax.fori_loop(0, C, body, 0)
    plsc.subcore_barrier()
    pltpu.sync_copy(deg_sp.at[pl.ds(s * TPR, TPR)],
                    out_hbm.at[c, pl.ds(s * TPR, TPR)])

  degp = deg_kernel(dstp)
  deg_t = degp.T

  MB = 512

  def mm_body(x_ref, w_ref, b_ref, dg_ref, g_ref):
    d = dg_ref[:, 0:1] + dg_ref[:, 1:2] + 1.0
    dinv = lax.rsqrt(jnp.maximum(d, 1.0))
    h = jnp.dot(x_ref[...], w_ref[...],
                preferred_element_type=jnp.float32) + b_ref[...]
    g_ref[...] = h * dinv

  g = pl.pallas_call(
      mm_body,
      grid=(NPAD // MB,),
      in_specs=[
          pl.BlockSpec((MB, Din), lambda i: (i, 0)),
          pl.BlockSpec((Din, Dout), lambda i: (0, 0)),
          pl.BlockSpec((1, Dout), lambda i: (0, 0)),
          pl.BlockSpec((MB, NC), lambda i: (i, 0)),
      ],
      out_specs=pl.BlockSpec((MB, Dout), lambda i: (i, 0)),
      out_shape=jax.ShapeDtypeStruct((NPAD, Dout), jnp.float32),
  )(Xp, W, b.reshape(1, Dout), deg_t)

  @functools.partial(
      pl.kernel,
      out_type=jax.ShapeDtypeStruct((NC, NPAD, Dout), jnp.float32),
      mesh=_sc_mesh(),
      scratch_types=[
          pltpu.VMEM((C, K), jnp.int32),
          pltpu.VMEM((C, K), jnp.int32),
          pltpu.VMEM((K, Dout), jnp.float32),
          pltpu.VMEM_SHARED((NPAD, Dout), jnp.float32),
          pltpu.SemaphoreType.DMA,
      ],
  )
  def scat_kernel(g_hbm, src_hbm, dst_hbm, out_hbm,
                  si_v, di_v, rows_v, acc_sp, sem):
    c = lax.axis_index("c")
    s = lax.axis_index("s")
    wid = c * NS + s
    pltpu.sync_copy(src_hbm.at[wid], si_v)
    pltpu.sync_copy(dst_hbm.at[wid], di_v)

    def zbody(i, carry):
      for jj in range(Dout // L):
        rows_v[i, pl.ds(jj * L, L)] = jnp.zeros((L,), jnp.float32)
      return carry

    lax.fori_loop(0, K, zbody, 0)
    for r in range(TPR // K):
      pltpu.sync_copy(rows_v, acc_sp.at[pl.ds(s * TPR + r * K, K)])
    plsc.subcore_barrier()

    def body(j, carry):
      pltpu.async_copy(g_hbm.at[si_v.at[j]], rows_v, sem).wait()
      pltpu.sync_copy(rows_v, acc_sp.at[di_v.at[j]], add=True)
      return carry

    lax.fori_loop(0, C, body, 0)
    plsc.subcore_barrier()
    for r in range(TPR // K):
      pltpu.sync_copy(acc_sp.at[pl.ds(s * TPR + r * K, K)],
                      out_hbm.at[c, pl.ds(s * TPR + r * K, K)])

  accp = scat_kernel(g, srcp, dstp)

  MB2 = 1000

  def fin_body(a0_ref, a1_ref, g_ref, dg_ref, o_ref):
    d = dg_ref[:, 0:1] + dg_ref[:, 1:2] + 1.0
    dinv = lax.rsqrt(jnp.maximum(d, 1.0))
    tot = a0_ref[...] + a1_ref[...] + g_ref[...]
    o_ref[...] = jnp.maximum(tot * dinv, 0.0)

  out = pl.pallas_call(
      fin_body,
      grid=(N // MB2,),
      in_specs=[
          pl.BlockSpec((MB2, Dout), lambda i: (i, 0)),
          pl.BlockSpec((MB2, Dout), lambda i: (i, 0)),
          pl.BlockSpec((MB2, Dout), lambda i: (i, 0)),
          pl.BlockSpec((MB2, NC), lambda i: (i, 0)),
      ],
      out_specs=pl.BlockSpec((MB2, Dout), lambda i: (i, 0)),
      out_shape=jax.ShapeDtypeStruct((N, Dout), jnp.float32),
  )(accp[0], accp[1], g, deg_t)

  return out


# speedup vs baseline: 39.8903x; 1.3149x over previous
import functools

import jax
import jax.numpy as jnp
from jax import lax
from jax.experimental import pallas as pl
from jax.experimental.pallas import tpu as pltpu
from jax.experimental.pallas import tpu_sc as plsc

NC = 2
NS = 16
NW = NC * NS
L = 16
K = 128


def _sc_mesh():
  return plsc.VectorSubcoreMesh(core_axis_name="c", subcore_axis_name="s")


def kernel(X, edge_index, W, b):
  N, Din = X.shape
  Dout = W.shape[1]
  E = edge_index.shape[1]

  row_unit = NS * K
  NPAD = ((N + 1 + row_unit - 1) // row_unit) * row_unit
  TPR = NPAD // NS

  C = (((E + NW * K - 1) // (NW * K)) + 3) // 4 * 4
  EPW = C * K
  EPAD = EPW * NW
  H = 2
  CH = C // H

  src = edge_index[0].astype(jnp.int32)
  dst = edge_index[1].astype(jnp.int32)
  pad = N + jnp.arange(EPAD - E, dtype=jnp.int32) % (NPAD - N)
  srcp = jnp.concatenate([src, pad]).reshape(NW, C, K)
  dstp = jnp.concatenate([dst, pad]).reshape(NW, C, K)
  Xp = jnp.pad(X, ((0, NPAD - N), (0, 0)))

  @functools.partial(
      pl.kernel,
      out_type=jax.ShapeDtypeStruct((NC, NPAD), jnp.float32),
      mesh=_sc_mesh(),
      scratch_types=[
          pltpu.VMEM((C, K), jnp.int32),
          pltpu.VMEM((K,), jnp.float32),
          pltpu.VMEM((TPR,), jnp.float32),
          pltpu.VMEM_SHARED((NPAD,), jnp.float32),
      ],
  )
  def deg_kernel(dst_hbm, out_hbm, idx_v, ones_v, zero_v, deg_sp):
    c = lax.axis_index("c")
    s = lax.axis_index("s")
    wid = c * NS + s
    pltpu.sync_copy(dst_hbm.at[wid], idx_v)
    for i in range(K // L):
      ones_v[pl.ds(i * L, L)] = jnp.ones((L,), jnp.float32)
    for i in range(TPR // L):
      zero_v[pl.ds(i * L, L)] = jnp.zeros((L,), jnp.float32)
    pltpu.sync_copy(zero_v, deg_sp.at[pl.ds(s * TPR, TPR)])
    plsc.subcore_barrier()

    def body(j, carry):
      pltpu.sync_copy(ones_v, deg_sp.at[idx_v.at[j]], add=True)
      return carry

    lax.fori_loop(0, C, body, 0)
    plsc.subcore_barrier()
    pltpu.sync_copy(deg_sp.at[pl.ds(s * TPR, TPR)],
                    out_hbm.at[c, pl.ds(s * TPR, TPR)])

  degp = deg_kernel(dstp)
  deg_t = degp.T

  MB = 512

  def mm_body(x_ref, w_ref, b_ref, dg_ref, g_ref):
    d = dg_ref[:, 0:1] + dg_ref[:, 1:2] + 1.0
    dinv = lax.rsqrt(jnp.maximum(d, 1.0))
    h = jnp.dot(x_ref[...], w_ref[...],
                preferred_element_type=jnp.float32) + b_ref[...]
    g_ref[...] = h * dinv

  g = pl.pallas_call(
      mm_body,
      grid=(NPAD // MB,),
      in_specs=[
          pl.BlockSpec((MB, Din), lambda i: (i, 0)),
          pl.BlockSpec((Din, Dout), lambda i: (0, 0)),
          pl.BlockSpec((1, Dout), lambda i: (0, 0)),
          pl.BlockSpec((MB, NC), lambda i: (i, 0)),
      ],
      out_specs=pl.BlockSpec((MB, Dout), lambda i: (i, 0)),
      out_shape=jax.ShapeDtypeStruct((NPAD, Dout), jnp.float32),
  )(Xp, W, b.reshape(1, Dout), deg_t)

  @functools.partial(
      pl.kernel,
      out_type=jax.ShapeDtypeStruct((NC, NPAD, Dout), jnp.float32),
      mesh=_sc_mesh(),
      scratch_types=[
          pltpu.VMEM((CH, K), jnp.int32),
          pltpu.VMEM((CH, K), jnp.int32),
          pltpu.VMEM((K, Dout), jnp.float32),
          pltpu.VMEM((K, Dout), jnp.float32),
          pltpu.VMEM_SHARED((NPAD, Dout), jnp.float32),
          pltpu.SemaphoreType.DMA,
          pltpu.SemaphoreType.DMA,
      ],
  )
  def scat_kernel(g_hbm, src_hbm, dst_hbm, out_hbm,
                  si_v, di_v, rows_a, rows_b, acc_sp, sem_a, sem_b):
    c = lax.axis_index("c")
    s = lax.axis_index("s")
    wid = c * NS + s

    def zbody(i, carry):
      for jj in range(Dout // L):
        rows_a[i, pl.ds(jj * L, L)] = jnp.zeros((L,), jnp.float32)
      return carry

    lax.fori_loop(0, K, zbody, 0)
    for r in range(TPR // K):
      pltpu.sync_copy(rows_a, acc_sp.at[pl.ds(s * TPR + r * K, K)])
    plsc.subcore_barrier()

    for h in range(H):
      pltpu.sync_copy(src_hbm.at[wid, pl.ds(h * CH, CH)], si_v)
      pltpu.sync_copy(dst_hbm.at[wid, pl.ds(h * CH, CH)], di_v)
      pltpu.async_copy(g_hbm.at[si_v.at[0]], rows_a, sem_a)
      pltpu.async_copy(g_hbm.at[si_v.at[1]], rows_b, sem_b)

      def body(jj, carry):
        j = jj * 2
        for rows_v, sem, off in ((rows_a, sem_a, 0), (rows_b, sem_b, 1)):
          pltpu.make_async_copy(g_hbm.at[si_v.at[j + off]], rows_v, sem).wait()
          pltpu.sync_copy(rows_v, acc_sp.at[di_v.at[j + off]], add=True)
          nxt = jnp.minimum(j + off + 2, CH - 1)
          pltpu.async_copy(g_hbm.at[si_v.at[nxt]], rows_v, sem)
        return carry

      lax.fori_loop(0, CH // 2, body, 0)
      pltpu.make_async_copy(g_hbm.at[si_v.at[0]], rows_a, sem_a).wait()
      pltpu.make_async_copy(g_hbm.at[si_v.at[0]], rows_b, sem_b).wait()
    plsc.subcore_barrier()
    for r in range(TPR // K):
      pltpu.sync_copy(acc_sp.at[pl.ds(s * TPR + r * K, K)],
                      out_hbm.at[c, pl.ds(s * TPR + r * K, K)])

  accp = scat_kernel(g, srcp, dstp)

  MB2 = 1000

  def fin_body(a0_ref, a1_ref, g_ref, dg_ref, o_ref):
    d = dg_ref[:, 0:1] + dg_ref[:, 1:2] + 1.0
    dinv = lax.rsqrt(jnp.maximum(d, 1.0))
    tot = a0_ref[...] + a1_ref[...] + g_ref[...]
    o_ref[...] = jnp.maximum(tot * dinv, 0.0)

  out = pl.pallas_call(
      fin_body,
      grid=(N // MB2,),
      in_specs=[
          pl.BlockSpec((MB2, Dout), lambda i: (i, 0)),
          pl.BlockSpec((MB2, Dout), lambda i: (i, 0)),
          pl.BlockSpec((MB2, Dout), lambda i: (i, 0)),
          pl.BlockSpec((MB2, NC), lambda i: (i, 0)),
      ],
      out_specs=pl.BlockSpec((MB2, Dout), lambda i: (i, 0)),
      out_shape=jax.ShapeDtypeStruct((N, Dout), jnp.float32),
  )(accp[0], accp[1], g, deg_t)

  return out


# speedup vs baseline: 42.3128x; 1.0607x over previous
import functools

import jax
import jax.numpy as jnp
from jax import lax
from jax.experimental import pallas as pl
from jax.experimental.pallas import tpu as pltpu
from jax.experimental.pallas import tpu_sc as plsc

NC = 2
NS = 16
NW = NC * NS
L = 16
K = 128


def _sc_mesh():
  return plsc.VectorSubcoreMesh(core_axis_name="c", subcore_axis_name="s")


def kernel(X, edge_index, W, b):
  N, Din = X.shape
  Dout = W.shape[1]
  E = edge_index.shape[1]

  row_unit = NS * K
  NPAD = ((N + 1 + row_unit - 1) // row_unit) * row_unit
  TPR = NPAD // NS

  C = (((E + NW * K - 1) // (NW * K)) + 3) // 4 * 4
  EPW = C * K
  EPAD = EPW * NW
  H = 2
  CH = C // H

  src = edge_index[0].astype(jnp.int32)
  dst = edge_index[1].astype(jnp.int32)
  pad = N + jnp.arange(EPAD - E, dtype=jnp.int32) % (NPAD - N)
  srcp = jnp.concatenate([src, pad]).reshape(NW, C, K)
  dstp = jnp.concatenate([dst, pad]).reshape(NW, C, K)

  @functools.partial(
      pl.kernel,
      out_type=jax.ShapeDtypeStruct((NC, NPAD), jnp.float32),
      mesh=_sc_mesh(),
      scratch_types=[
          pltpu.VMEM((C, K), jnp.int32),
          pltpu.VMEM((K,), jnp.float32),
          pltpu.VMEM((TPR,), jnp.float32),
          pltpu.VMEM_SHARED((NPAD,), jnp.float32),
          pltpu.SemaphoreType.DMA,
      ],
  )
  def deg_kernel(dst_hbm, out_hbm, idx_v, ones_v, zero_v, deg_sp, sem):
    c = lax.axis_index("c")
    s = lax.axis_index("s")
    wid = c * NS + s
    pltpu.sync_copy(dst_hbm.at[wid], idx_v)
    for i in range(K // L):
      ones_v[pl.ds(i * L, L)] = jnp.ones((L,), jnp.float32)
    for i in range(TPR // L):
      zero_v[pl.ds(i * L, L)] = jnp.zeros((L,), jnp.float32)
    pltpu.sync_copy(zero_v, deg_sp.at[pl.ds(s * TPR, TPR)])
    plsc.subcore_barrier()

    def body(j, carry):
      pltpu.async_copy(ones_v, deg_sp.at[idx_v.at[j]], sem, add=True)
      return carry

    lax.fori_loop(0, C, body, 0)
    pltpu.make_async_copy(dst_hbm.at[wid], idx_v, sem).wait()
    plsc.subcore_barrier()
    pltpu.sync_copy(deg_sp.at[pl.ds(s * TPR, TPR)],
                    out_hbm.at[c, pl.ds(s * TPR, TPR)])

  degp = deg_kernel(dstp)
  deg_t = degp.T

  MB = 1000

  def mm_body(x_ref, w_ref, b_ref, dg_ref, g_ref):
    d = dg_ref[:, 0:1] + dg_ref[:, 1:2] + 1.0
    dinv = lax.rsqrt(jnp.maximum(d, 1.0))
    h = jnp.dot(x_ref[...], w_ref[...],
                preferred_element_type=jnp.float32) + b_ref[...]
    g_ref[...] = h * dinv

  g = pl.pallas_call(
      mm_body,
      grid=(N // MB,),
      in_specs=[
          pl.BlockSpec((MB, Din), lambda i: (i, 0)),
          pl.BlockSpec((Din, Dout), lambda i: (0, 0)),
          pl.BlockSpec((1, Dout), lambda i: (0, 0)),
          pl.BlockSpec((MB, NC), lambda i: (i, 0)),
      ],
      out_specs=pl.BlockSpec((MB, Dout), lambda i: (i, 0)),
      out_shape=jax.ShapeDtypeStruct((NPAD, Dout), jnp.float32),
  )(X, W, b.reshape(1, Dout), deg_t)

  @functools.partial(
      pl.kernel,
      out_type=jax.ShapeDtypeStruct((NC, NPAD, Dout), jnp.float32),
      mesh=_sc_mesh(),
      scratch_types=[
          pltpu.VMEM((CH, K), jnp.int32),
          pltpu.VMEM((CH, K), jnp.int32),
          pltpu.VMEM((K, Dout), jnp.float32),
          pltpu.VMEM((K, Dout), jnp.float32),
          pltpu.VMEM_SHARED((NPAD, Dout), jnp.float32),
          pltpu.SemaphoreType.DMA,
          pltpu.SemaphoreType.DMA,
      ],
  )
  def scat_kernel(g_hbm, src_hbm, dst_hbm, out_hbm,
                  si_v, di_v, rows_a, rows_b, acc_sp, sem_a, sem_b):
    c = lax.axis_index("c")
    s = lax.axis_index("s")
    wid = c * NS + s

    def zbody(i, carry):
      for jj in range(Dout // L):
        rows_a[i, pl.ds(jj * L, L)] = jnp.zeros((L,), jnp.float32)
      return carry

    lax.fori_loop(0, K, zbody, 0)
    for r in range(TPR // K):
      pltpu.sync_copy(rows_a, acc_sp.at[pl.ds(s * TPR + r * K, K)])
    plsc.subcore_barrier()

    for h in range(H):
      pltpu.sync_copy(src_hbm.at[wid, pl.ds(h * CH, CH)], si_v)
      pltpu.sync_copy(dst_hbm.at[wid, pl.ds(h * CH, CH)], di_v)
      pltpu.async_copy(g_hbm.at[si_v.at[0]], rows_a, sem_a)
      pltpu.async_copy(g_hbm.at[si_v.at[1]], rows_b, sem_b)

      def body(jj, carry):
        j = jj * 2
        for rows_v, sem, off in ((rows_a, sem_a, 0), (rows_b, sem_b, 1)):
          pltpu.make_async_copy(g_hbm.at[si_v.at[j + off]], rows_v, sem).wait()
          pltpu.sync_copy(rows_v, acc_sp.at[di_v.at[j + off]], add=True)
          nxt = jnp.minimum(j + off + 2, CH - 1)
          pltpu.async_copy(g_hbm.at[si_v.at[nxt]], rows_v, sem)
        return carry

      lax.fori_loop(0, CH // 2, body, 0)
      pltpu.make_async_copy(g_hbm.at[si_v.at[0]], rows_a, sem_a).wait()
      pltpu.make_async_copy(g_hbm.at[si_v.at[0]], rows_b, sem_b).wait()
    plsc.subcore_barrier()
    for r in range(TPR // K):
      pltpu.sync_copy(acc_sp.at[pl.ds(s * TPR + r * K, K)],
                      out_hbm.at[c, pl.ds(s * TPR + r * K, K)])

  accp = scat_kernel(g, srcp, dstp)

  MB2 = 1000

  def fin_body(a0_ref, a1_ref, g_ref, dg_ref, o_ref):
    d = dg_ref[:, 0:1] + dg_ref[:, 1:2] + 1.0
    dinv = lax.rsqrt(jnp.maximum(d, 1.0))
    tot = a0_ref[...] + a1_ref[...] + g_ref[...]
    o_ref[...] = jnp.maximum(tot * dinv, 0.0)

  out = pl.pallas_call(
      fin_body,
      grid=(N // MB2,),
      in_specs=[
          pl.BlockSpec((MB2, Dout), lambda i: (i, 0)),
          pl.BlockSpec((MB2, Dout), lambda i: (i, 0)),
          pl.BlockSpec((MB2, Dout), lambda i: (i, 0)),
          pl.BlockSpec((MB2, NC), lambda i: (i, 0)),
      ],
      out_specs=pl.BlockSpec((MB2, Dout), lambda i: (i, 0)),
      out_shape=jax.ShapeDtypeStruct((N, Dout), jnp.float32),
  )(accp[0], accp[1], g, deg_t)

  return out


# speedup vs baseline: 44.7153x; 1.0568x over previous
import functools

import jax
import jax.numpy as jnp
from jax import lax
from jax.experimental import pallas as pl
from jax.experimental.pallas import tpu as pltpu
from jax.experimental.pallas import tpu_sc as plsc

NC = 2
NS = 16
NW = NC * NS
L = 16
K = 128


def _sc_mesh():
  return plsc.VectorSubcoreMesh(core_axis_name="c", subcore_axis_name="s")


def kernel(X, edge_index, W, b):
  N, Din = X.shape
  Dout = W.shape[1]
  E = edge_index.shape[1]

  row_unit = NS * K
  NPAD = ((N + 1 + row_unit - 1) // row_unit) * row_unit
  TPR = NPAD // NS

  C = (((E + NW * K - 1) // (NW * K)) + 3) // 4 * 4
  EPW = C * K
  EPAD = EPW * NW
  H = 2
  CH = C // H

  src = edge_index[0].astype(jnp.int32)
  dst = edge_index[1].astype(jnp.int32)
  pad = N + jnp.arange(EPAD - E, dtype=jnp.int32) % (NPAD - N)
  padb = jnp.broadcast_to(pad, (2, EPAD - E))
  ep = jnp.concatenate([edge_index.astype(jnp.int32), padb],
                       axis=1).reshape(2, NW, C, K)

  @functools.partial(
      pl.kernel,
      out_type=jax.ShapeDtypeStruct((NC, NPAD), jnp.float32),
      mesh=_sc_mesh(),
      scratch_types=[
          pltpu.VMEM((C, K), jnp.int32),
          pltpu.VMEM((K,), jnp.float32),
          pltpu.VMEM_SHARED((NPAD,), jnp.float32),
          pltpu.SemaphoreType.DMA,
      ],
  )
  def deg_kernel(ep_hbm, ones_hbm, zeros_hbm, out_hbm, idx_v, ones_v,
                 deg_sp, sem):
    c = lax.axis_index("c")
    s = lax.axis_index("s")
    wid = c * NS + s
    pltpu.sync_copy(ep_hbm.at[1, wid], idx_v)
    pltpu.sync_copy(ones_hbm, ones_v)
    pltpu.sync_copy(zeros_hbm, deg_sp.at[pl.ds(s * TPR, TPR)])
    plsc.subcore_barrier()

    def body(j, carry):
      pltpu.async_copy(ones_v, deg_sp.at[idx_v.at[j]], sem, add=True)
      return carry

    lax.fori_loop(0, C, body, 0)
    pltpu.make_async_copy(ep_hbm.at[1, wid], idx_v, sem).wait()
    plsc.subcore_barrier()
    pltpu.sync_copy(deg_sp.at[pl.ds(s * TPR, TPR)],
                    out_hbm.at[c, pl.ds(s * TPR, TPR)])

  ones_col = jnp.ones((K,), jnp.float32)
  zeros_col = jnp.zeros((TPR,), jnp.float32)
  degp = deg_kernel(ep, ones_col, zeros_col)
  deg_t = degp.T

  MB = 1000

  def mm_body(x_ref, w_ref, b_ref, h_ref):
    h_ref[...] = jnp.dot(x_ref[...], w_ref[...],
                         preferred_element_type=jnp.float32) + b_ref[...]

  hmat = pl.pallas_call(
      mm_body,
      grid=(N // MB,),
      in_specs=[
          pl.BlockSpec((MB, Din), lambda i: (i, 0)),
          pl.BlockSpec((Din, Dout), lambda i: (0, 0)),
          pl.BlockSpec((1, Dout), lambda i: (0, 0)),
      ],
      out_specs=pl.BlockSpec((MB, Dout), lambda i: (i, 0)),
      out_shape=jax.ShapeDtypeStruct((N, Dout), jnp.float32),
  )(X, W, b.reshape(1, Dout))

  def scale_body(h_ref, dg_ref, g_ref):
    d = dg_ref[:, 0:1] + dg_ref[:, 1:2] + 1.0
    g_ref[...] = h_ref[...] * lax.rsqrt(jnp.maximum(d, 1.0))

  g = pl.pallas_call(
      scale_body,
      grid=(N // MB,),
      in_specs=[
          pl.BlockSpec((MB, Dout), lambda i: (i, 0)),
          pl.BlockSpec((MB, NC), lambda i: (i, 0)),
      ],
      out_specs=pl.BlockSpec((MB, Dout), lambda i: (i, 0)),
      out_shape=jax.ShapeDtypeStruct((NPAD, Dout), jnp.float32),
  )(hmat, deg_t)

  @functools.partial(
      pl.kernel,
      out_type=jax.ShapeDtypeStruct((NC, NPAD, Dout), jnp.float32),
      mesh=_sc_mesh(),
      scratch_types=[
          pltpu.VMEM((CH, K), jnp.int32),
          pltpu.VMEM((CH, K), jnp.int32),
          pltpu.VMEM((K, Dout), jnp.float32),
          pltpu.VMEM((K, Dout), jnp.float32),
          pltpu.VMEM_SHARED((NPAD, Dout), jnp.float32),
          pltpu.SemaphoreType.DMA,
          pltpu.SemaphoreType.DMA,
      ],
  )
  def scat_kernel(g_hbm, ep_hbm, out_hbm,
                  si_v, di_v, rows_a, rows_b, acc_sp, sem_a, sem_b):
    c = lax.axis_index("c")
    s = lax.axis_index("s")
    wid = c * NS + s

    def zbody(i, carry):
      for jj in range(Dout // L):
        rows_a[i, pl.ds(jj * L, L)] = jnp.zeros((L,), jnp.float32)
      return carry

    lax.fori_loop(0, K, zbody, 0)
    for r in range(TPR // K):
      pltpu.sync_copy(rows_a, acc_sp.at[pl.ds(s * TPR + r * K, K)])
    plsc.subcore_barrier()

    for h in range(H):
      pltpu.sync_copy(ep_hbm.at[0, wid, pl.ds(h * CH, CH)], si_v)
      pltpu.sync_copy(ep_hbm.at[1, wid, pl.ds(h * CH, CH)], di_v)
      pltpu.async_copy(g_hbm.at[si_v.at[0]], rows_a, sem_a)
      pltpu.async_copy(g_hbm.at[si_v.at[1]], rows_b, sem_b)

      def body(jj, carry):
        j = jj * 2
        for rows_v, sem, off in ((rows_a, sem_a, 0), (rows_b, sem_b, 1)):
          pltpu.make_async_copy(g_hbm.at[si_v.at[j + off]], rows_v, sem).wait()
          pltpu.sync_copy(rows_v, acc_sp.at[di_v.at[j + off]], add=True)
          nxt = jnp.minimum(j + off + 2, CH - 1)
          pltpu.async_copy(g_hbm.at[si_v.at[nxt]], rows_v, sem)
        return carry

      lax.fori_loop(0, CH // 2, body, 0)
      pltpu.make_async_copy(g_hbm.at[si_v.at[0]], rows_a, sem_a).wait()
      pltpu.make_async_copy(g_hbm.at[si_v.at[0]], rows_b, sem_b).wait()
    plsc.subcore_barrier()
    for r in range(TPR // K):
      pltpu.sync_copy(acc_sp.at[pl.ds(s * TPR + r * K, K)],
                      out_hbm.at[c, pl.ds(s * TPR + r * K, K)])

  accp = scat_kernel(g, ep)

  MB2 = 1000

  def fin_body(a0_ref, a1_ref, g_ref, dg_ref, o_ref):
    d = dg_ref[:, 0:1] + dg_ref[:, 1:2] + 1.0
    dinv = lax.rsqrt(jnp.maximum(d, 1.0))
    tot = a0_ref[0] + a1_ref[0] + g_ref[...]
    o_ref[...] = jnp.maximum(tot * dinv, 0.0)

  out = pl.pallas_call(
      fin_body,
      grid=(N // MB2,),
      in_specs=[
          pl.BlockSpec((1, MB2, Dout), lambda i: (0, i, 0)),
          pl.BlockSpec((1, MB2, Dout), lambda i: (1, i, 0)),
          pl.BlockSpec((MB2, Dout), lambda i: (i, 0)),
          pl.BlockSpec((MB2, NC), lambda i: (i, 0)),
      ],
      out_specs=pl.BlockSpec((MB2, Dout), lambda i: (i, 0)),
      out_shape=jax.ShapeDtypeStruct((N, Dout), jnp.float32),
  )(accp, accp, g, deg_t)

  return out


# speedup vs baseline: 46.1179x; 1.0314x over previous
import functools

import jax
import jax.numpy as jnp
from jax import lax
from jax.experimental import pallas as pl
from jax.experimental.pallas import tpu as pltpu
from jax.experimental.pallas import tpu_sc as plsc

NC = 2
NS = 16
NW = NC * NS
L = 16
K = 128


def _sc_mesh():
  return plsc.VectorSubcoreMesh(core_axis_name="c", subcore_axis_name="s")


def kernel(X, edge_index, W, b):
  N, Din = X.shape
  Dout = W.shape[1]
  E = edge_index.shape[1]

  row_unit = NS * K
  NPAD = ((N + 1 + row_unit - 1) // row_unit) * row_unit
  TPR = NPAD // NS

  C = (((E + NW * K - 1) // (NW * K)) + 3) // 4 * 4
  EPW = C * K
  EPAD = EPW * NW
  H = 2
  CH = C // H

  src = edge_index[0].astype(jnp.int32)
  dst = edge_index[1].astype(jnp.int32)
  pad = N + jnp.arange(EPAD - E, dtype=jnp.int32) % (NPAD - N)
  padb = jnp.broadcast_to(pad, (2, EPAD - E))
  ep = jnp.concatenate([edge_index.astype(jnp.int32), padb],
                       axis=1).reshape(2, NW, C, K)

  @functools.partial(
      pl.kernel,
      out_type=jax.ShapeDtypeStruct((NC, NPAD), jnp.float32),
      mesh=_sc_mesh(),
      scratch_types=[
          pltpu.VMEM((C, K), jnp.int32),
          pltpu.VMEM((K,), jnp.float32),
          pltpu.VMEM_SHARED((NPAD,), jnp.float32),
          pltpu.SemaphoreType.DMA,
      ],
  )
  def deg_kernel(ep_hbm, ones_hbm, zeros_hbm, out_hbm, idx_v, ones_v,
                 deg_sp, sem):
    c = lax.axis_index("c")
    s = lax.axis_index("s")
    wid = c * NS + s
    pltpu.sync_copy(ep_hbm.at[1, wid], idx_v)
    pltpu.sync_copy(ones_hbm, ones_v)
    pltpu.sync_copy(zeros_hbm, deg_sp.at[pl.ds(s * TPR, TPR)])
    plsc.subcore_barrier()

    def body(j, carry):
      pltpu.async_copy(ones_v, deg_sp.at[idx_v.at[j]], sem, add=True)
      return carry

    lax.fori_loop(0, C, body, 0)
    pltpu.make_async_copy(ep_hbm.at[1, wid], idx_v, sem).wait()
    plsc.subcore_barrier()
    pltpu.sync_copy(deg_sp.at[pl.ds(s * TPR, TPR)],
                    out_hbm.at[c, pl.ds(s * TPR, TPR)])

  ones_col = jnp.ones((K,), jnp.float32)
  zeros_col = jnp.zeros((TPR,), jnp.float32)
  degp = deg_kernel(ep, ones_col, zeros_col)

  MB = 1024

  def mm_body(x_ref, w_ref, b_ref, h_ref):
    h_ref[...] = jnp.dot(x_ref[...], w_ref[...],
                         preferred_element_type=jnp.float32) + b_ref[...]

  hmat = pl.pallas_call(
      mm_body,
      grid=((N + MB - 1) // MB,),
      in_specs=[
          pl.BlockSpec((MB, Din), lambda i: (i, 0)),
          pl.BlockSpec((Din, Dout), lambda i: (0, 0)),
          pl.BlockSpec((1, Dout), lambda i: (0, 0)),
      ],
      out_specs=pl.BlockSpec((MB, Dout), lambda i: (i, 0)),
      out_shape=jax.ShapeDtypeStruct((N, Dout), jnp.float32),
  )(X, W, b.reshape(1, Dout))

  def scale_body(h_ref, dg_ref, g_ref):
    d = dg_ref[0:1, :] + dg_ref[1:2, :] + 1.0
    dinv = jnp.transpose(lax.rsqrt(jnp.maximum(d, 1.0)), (1, 0))
    g_ref[...] = h_ref[...] * dinv

  g = pl.pallas_call(
      scale_body,
      grid=((N + MB - 1) // MB,),
      in_specs=[
          pl.BlockSpec((MB, Dout), lambda i: (i, 0)),
          pl.BlockSpec((NC, MB), lambda i: (0, i)),
      ],
      out_specs=pl.BlockSpec((MB, Dout), lambda i: (i, 0)),
      out_shape=jax.ShapeDtypeStruct((NPAD, Dout), jnp.float32),
  )(hmat, degp)

  @functools.partial(
      pl.kernel,
      out_type=jax.ShapeDtypeStruct((NC, NPAD, Dout), jnp.float32),
      mesh=_sc_mesh(),
      scratch_types=[
          pltpu.VMEM((CH, K), jnp.int32),
          pltpu.VMEM((CH, K), jnp.int32),
          pltpu.VMEM((K, Dout), jnp.float32),
          pltpu.VMEM((K, Dout), jnp.float32),
          pltpu.VMEM_SHARED((NPAD, Dout), jnp.float32),
          pltpu.SemaphoreType.DMA,
          pltpu.SemaphoreType.DMA,
      ],
  )
  def scat_kernel(g_hbm, ep_hbm, out_hbm,
                  si_v, di_v, rows_a, rows_b, acc_sp, sem_a, sem_b):
    c = lax.axis_index("c")
    s = lax.axis_index("s")
    wid = c * NS + s

    def zbody(i, carry):
      for jj in range(Dout // L):
        rows_a[i, pl.ds(jj * L, L)] = jnp.zeros((L,), jnp.float32)
      return carry

    lax.fori_loop(0, K, zbody, 0)
    for r in range(TPR // K):
      pltpu.sync_copy(rows_a, acc_sp.at[pl.ds(s * TPR + r * K, K)])
    plsc.subcore_barrier()

    for h in range(H):
      pltpu.sync_copy(ep_hbm.at[0, wid, pl.ds(h * CH, CH)], si_v)
      pltpu.sync_copy(ep_hbm.at[1, wid, pl.ds(h * CH, CH)], di_v)
      pltpu.async_copy(g_hbm.at[si_v.at[0]], rows_a, sem_a)
      pltpu.async_copy(g_hbm.at[si_v.at[1]], rows_b, sem_b)

      def body(jj, carry):
        j = jj * 2
        for rows_v, sem, off in ((rows_a, sem_a, 0), (rows_b, sem_b, 1)):
          pltpu.make_async_copy(g_hbm.at[si_v.at[j + off]], rows_v, sem).wait()
          pltpu.sync_copy(rows_v, acc_sp.at[di_v.at[j + off]], add=True)
          nxt = jnp.minimum(j + off + 2, CH - 1)
          pltpu.async_copy(g_hbm.at[si_v.at[nxt]], rows_v, sem)
        return carry

      lax.fori_loop(0, CH // 2, body, 0)
      pltpu.make_async_copy(g_hbm.at[si_v.at[0]], rows_a, sem_a).wait()
      pltpu.make_async_copy(g_hbm.at[si_v.at[0]], rows_b, sem_b).wait()
    plsc.subcore_barrier()
    for r in range(TPR // K):
      pltpu.sync_copy(acc_sp.at[pl.ds(s * TPR + r * K, K)],
                      out_hbm.at[c, pl.ds(s * TPR + r * K, K)])

  accp = scat_kernel(g, ep)

  MB2 = 1024

  def fin_body(a0_ref, a1_ref, g_ref, dg_ref, o_ref):
    d = dg_ref[0:1, :] + dg_ref[1:2, :] + 1.0
    dinv = jnp.transpose(lax.rsqrt(jnp.maximum(d, 1.0)), (1, 0))
    tot = a0_ref[0] + a1_ref[0] + g_ref[...]
    o_ref[...] = jnp.maximum(tot * dinv, 0.0)

  out = pl.pallas_call(
      fin_body,
      grid=((N + MB2 - 1) // MB2,),
      in_specs=[
          pl.BlockSpec((1, MB2, Dout), lambda i: (0, i, 0)),
          pl.BlockSpec((1, MB2, Dout), lambda i: (1, i, 0)),
          pl.BlockSpec((MB2, Dout), lambda i: (i, 0)),
          pl.BlockSpec((NC, MB2), lambda i: (0, i)),
      ],
      out_specs=pl.BlockSpec((MB2, Dout), lambda i: (i, 0)),
      out_shape=jax.ShapeDtypeStruct((N, Dout), jnp.float32),
  )(accp, accp, g, degp)

  return out


# speedup vs baseline: 48.3572x; 1.0486x over previous
import functools

import jax
import jax.numpy as jnp
from jax import lax
from jax.experimental import pallas as pl
from jax.experimental.pallas import tpu as pltpu
from jax.experimental.pallas import tpu_sc as plsc

NC = 2
NS = 16
NW = NC * NS
L = 16
K = 128


def _sc_mesh():
  return plsc.VectorSubcoreMesh(core_axis_name="c", subcore_axis_name="s")


def kernel(X, edge_index, W, b):
  N, Din = X.shape
  Dout = W.shape[1]
  E = edge_index.shape[1]

  row_unit = NS * K
  NPAD = ((N + 1 + row_unit - 1) // row_unit) * row_unit
  TPR = NPAD // NS

  C = (((E + NW * K - 1) // (NW * K)) + 3) // 4 * 4
  EPW = C * K
  EPAD = EPW * NW
  H = 2
  CH = C // H

  src = edge_index[0].astype(jnp.int32)
  dst = edge_index[1].astype(jnp.int32)
  pad = N + jnp.arange(EPAD - E, dtype=jnp.int32) % (NPAD - N)
  padb = jnp.broadcast_to(pad, (2, EPAD - E))
  ep = jnp.concatenate([edge_index.astype(jnp.int32), padb],
                       axis=1).reshape(2, NW, C, K)

  @functools.partial(
      pl.kernel,
      out_type=jax.ShapeDtypeStruct((NC, NPAD), jnp.float32),
      mesh=_sc_mesh(),
      scratch_types=[
          pltpu.VMEM((C, K), jnp.int32),
          pltpu.VMEM((K,), jnp.float32),
          pltpu.VMEM((TPR,), jnp.float32),
          pltpu.VMEM_SHARED((NPAD,), jnp.float32),
          pltpu.SemaphoreType.DMA,
      ],
  )
  def deg_kernel(ep_hbm, out_hbm, idx_v, ones_v, zero_v, deg_sp, sem):
    c = lax.axis_index("c")
    s = lax.axis_index("s")
    wid = c * NS + s
    pltpu.sync_copy(ep_hbm.at[1, wid], idx_v)
    for i in range(K // L):
      ones_v[pl.ds(i * L, L)] = jnp.ones((L,), jnp.float32)
    for i in range(TPR // L):
      zero_v[pl.ds(i * L, L)] = jnp.zeros((L,), jnp.float32)
    pltpu.sync_copy(zero_v, deg_sp.at[pl.ds(s * TPR, TPR)])
    plsc.subcore_barrier()

    def body(j, carry):
      pltpu.async_copy(ones_v, deg_sp.at[idx_v.at[j]], sem, add=True)
      return carry

    lax.fori_loop(0, C, body, 0)
    pltpu.make_async_copy(ep_hbm.at[1, wid], idx_v, sem).wait()
    plsc.subcore_barrier()
    pltpu.sync_copy(deg_sp.at[pl.ds(s * TPR, TPR)],
                    out_hbm.at[c, pl.ds(s * TPR, TPR)])

  degp = deg_kernel(ep)

  MB = 2048

  def mm_body(x_ref, w_ref, b_ref, h_ref):
    h_ref[...] = jnp.dot(x_ref[...], w_ref[...],
                         preferred_element_type=jnp.float32) + b_ref[...]

  hmat = pl.pallas_call(
      mm_body,
      grid=((N + MB - 1) // MB,),
      in_specs=[
          pl.BlockSpec((MB, Din), lambda i: (i, 0)),
          pl.BlockSpec((Din, Dout), lambda i: (0, 0)),
          pl.BlockSpec((1, Dout), lambda i: (0, 0)),
      ],
      out_specs=pl.BlockSpec((MB, Dout), lambda i: (i, 0)),
      out_shape=jax.ShapeDtypeStruct((N, Dout), jnp.float32),
  )(X, W, b.reshape(1, Dout))

  def scale_body(h_ref, dg_ref, g_ref):
    d = dg_ref[0:1, :] + dg_ref[1:2, :] + 1.0
    dinv = jnp.transpose(lax.rsqrt(jnp.maximum(d, 1.0)), (1, 0))
    g_ref[...] = h_ref[...] * dinv

  g = pl.pallas_call(
      scale_body,
      grid=((N + MB - 1) // MB,),
      in_specs=[
          pl.BlockSpec((MB, Dout), lambda i: (i, 0)),
          pl.BlockSpec((NC, MB), lambda i: (0, i)),
      ],
      out_specs=pl.BlockSpec((MB, Dout), lambda i: (i, 0)),
      out_shape=jax.ShapeDtypeStruct((NPAD, Dout), jnp.float32),
  )(hmat, degp)

  @functools.partial(
      pl.kernel,
      out_type=jax.ShapeDtypeStruct((NC, NPAD, Dout), jnp.float32),
      mesh=_sc_mesh(),
      scratch_types=[
          pltpu.VMEM((CH, K), jnp.int32),
          pltpu.VMEM((CH, K), jnp.int32),
          pltpu.VMEM((K, Dout), jnp.float32),
          pltpu.VMEM((K, Dout), jnp.float32),
          pltpu.VMEM_SHARED((NPAD, Dout), jnp.float32),
          pltpu.SemaphoreType.DMA,
          pltpu.SemaphoreType.DMA,
      ],
  )
  def scat_kernel(g_hbm, ep_hbm, out_hbm,
                  si_v, di_v, rows_a, rows_b, acc_sp, sem_a, sem_b):
    c = lax.axis_index("c")
    s = lax.axis_index("s")
    wid = c * NS + s

    def zbody(i, carry):
      for jj in range(Dout // L):
        rows_a[i, pl.ds(jj * L, L)] = jnp.zeros((L,), jnp.float32)
      return carry

    lax.fori_loop(0, K, zbody, 0)
    for r in range(TPR // K):
      pltpu.sync_copy(rows_a, acc_sp.at[pl.ds(s * TPR + r * K, K)])
    plsc.subcore_barrier()

    for h in range(H):
      pltpu.sync_copy(ep_hbm.at[0, wid, pl.ds(h * CH, CH)], si_v)
      pltpu.sync_copy(ep_hbm.at[1, wid, pl.ds(h * CH, CH)], di_v)
      pltpu.async_copy(g_hbm.at[si_v.at[0]], rows_a, sem_a)
      pltpu.async_copy(g_hbm.at[si_v.at[1]], rows_b, sem_b)

      def body(jj, carry):
        j = jj * 2
        for rows_v, sem, off in ((rows_a, sem_a, 0), (rows_b, sem_b, 1)):
          pltpu.make_async_copy(g_hbm.at[si_v.at[j + off]], rows_v, sem).wait()
          pltpu.sync_copy(rows_v, acc_sp.at[di_v.at[j + off]], add=True)
          nxt = jnp.minimum(j + off + 2, CH - 1)
          pltpu.async_copy(g_hbm.at[si_v.at[nxt]], rows_v, sem)
        return carry

      lax.fori_loop(0, CH // 2, body, 0)
      pltpu.make_async_copy(g_hbm.at[si_v.at[0]], rows_a, sem_a).wait()
      pltpu.make_async_copy(g_hbm.at[si_v.at[0]], rows_b, sem_b).wait()
    plsc.subcore_barrier()
    for r in range(TPR // K):
      pltpu.sync_copy(acc_sp.at[pl.ds(s * TPR + r * K, K)],
                      out_hbm.at[c, pl.ds(s * TPR + r * K, K)])

  accp = scat_kernel(g, ep)

  MB2 = 2048

  def fin_body(a0_ref, a1_ref, g_ref, dg_ref, o_ref):
    d = dg_ref[0:1, :] + dg_ref[1:2, :] + 1.0
    dinv = jnp.transpose(lax.rsqrt(jnp.maximum(d, 1.0)), (1, 0))
    tot = a0_ref[0] + a1_ref[0] + g_ref[...]
    o_ref[...] = jnp.maximum(tot * dinv, 0.0)

  out = pl.pallas_call(
      fin_body,
      grid=((N + MB2 - 1) // MB2,),
      in_specs=[
          pl.BlockSpec((1, MB2, Dout), lambda i: (0, i, 0)),
          pl.BlockSpec((1, MB2, Dout), lambda i: (1, i, 0)),
          pl.BlockSpec((MB2, Dout), lambda i: (i, 0)),
          pl.BlockSpec((NC, MB2), lambda i: (0, i)),
      ],
      out_specs=pl.BlockSpec((MB2, Dout), lambda i: (i, 0)),
      out_shape=jax.ShapeDtypeStruct((N, Dout), jnp.float32),
  )(accp, accp, g, degp)

  return out


# speedup vs baseline: 50.4023x; 1.0423x over previous
import functools

import jax
import jax.numpy as jnp
from jax import lax
from jax.experimental import pallas as pl
from jax.experimental.pallas import tpu as pltpu
from jax.experimental.pallas import tpu_sc as plsc

NC = 2
NS = 16
NW = NC * NS
L = 16
K = 128


def _sc_mesh():
  return plsc.VectorSubcoreMesh(core_axis_name="c", subcore_axis_name="s")


def kernel(X, edge_index, W, b):
  N, Din = X.shape
  Dout = W.shape[1]
  E = edge_index.shape[1]

  row_unit = NS * K
  NPAD = ((N + 1 + row_unit - 1) // row_unit) * row_unit
  TPR = NPAD // NS

  C = (((E + NW * K - 1) // (NW * K)) + 3) // 4 * 4
  EPW = C * K
  EPAD = EPW * NW
  H = 2
  CH = C // H

  ei = edge_index.astype(jnp.int32)
  last = (NW - 1) * EPW
  pad = N + jnp.arange(EPAD - E, dtype=jnp.int32) % (NPAD - N)
  padb = jnp.broadcast_to(pad, (2, EPAD - E))
  tail = jnp.concatenate([ei[:, last:], padb], axis=1)

  @functools.partial(
      pl.kernel,
      out_type=jax.ShapeDtypeStruct((NC, NPAD), jnp.float32),
      mesh=_sc_mesh(),
      scratch_types=[
          pltpu.VMEM((EPW,), jnp.int32),
          pltpu.VMEM((K,), jnp.float32),
          pltpu.VMEM((TPR,), jnp.float32),
          pltpu.VMEM_SHARED((NPAD,), jnp.float32),
          pltpu.SemaphoreType.DMA,
      ],
  )
  def deg_kernel(ei_hbm, tail_hbm, out_hbm, idx_v, ones_v, zero_v, deg_sp,
                 sem):
    c = lax.axis_index("c")
    s = lax.axis_index("s")
    wid = c * NS + s
    @pl.when(wid < NW - 1)
    def _():
      pltpu.sync_copy(ei_hbm.at[1, pl.ds(wid * EPW, EPW)], idx_v)

    @pl.when(wid == NW - 1)
    def _():
      pltpu.sync_copy(tail_hbm.at[1], idx_v)

    for i in range(K // L):
      ones_v[pl.ds(i * L, L)] = jnp.ones((L,), jnp.float32)
    for i in range(TPR // L):
      zero_v[pl.ds(i * L, L)] = jnp.zeros((L,), jnp.float32)
    pltpu.sync_copy(zero_v, deg_sp.at[pl.ds(s * TPR, TPR)])
    plsc.subcore_barrier()

    def body(j, carry):
      pltpu.async_copy(ones_v, deg_sp.at[idx_v.at[pl.ds(j * K, K)]], sem,
                       add=True)
      return carry

    lax.fori_loop(0, C, body, 0)
    pltpu.make_async_copy(tail_hbm.at[1], idx_v, sem).wait()
    plsc.subcore_barrier()
    pltpu.sync_copy(deg_sp.at[pl.ds(s * TPR, TPR)],
                    out_hbm.at[c, pl.ds(s * TPR, TPR)])

  degp = deg_kernel(ei, tail)

  MB = 2048

  def mm_body(x_ref, w_ref, b_ref, h_ref):
    h_ref[...] = jnp.dot(x_ref[...], w_ref[...],
                         preferred_element_type=jnp.float32) + b_ref[...]

  hmat = pl.pallas_call(
      mm_body,
      grid=((N + MB - 1) // MB,),
      in_specs=[
          pl.BlockSpec((MB, Din), lambda i: (i, 0)),
          pl.BlockSpec((Din, Dout), lambda i: (0, 0)),
          pl.BlockSpec((1, Dout), lambda i: (0, 0)),
      ],
      out_specs=pl.BlockSpec((MB, Dout), lambda i: (i, 0)),
      out_shape=jax.ShapeDtypeStruct((N, Dout), jnp.float32),
  )(X, W, b.reshape(1, Dout))

  def scale_body(h_ref, dg_ref, g_ref):
    d = dg_ref[0:1, :] + dg_ref[1:2, :] + 1.0
    dinv = jnp.transpose(lax.rsqrt(jnp.maximum(d, 1.0)), (1, 0))
    g_ref[...] = h_ref[...] * dinv

  g = pl.pallas_call(
      scale_body,
      grid=((N + MB - 1) // MB,),
      in_specs=[
          pl.BlockSpec((MB, Dout), lambda i: (i, 0)),
          pl.BlockSpec((NC, MB), lambda i: (0, i)),
      ],
      out_specs=pl.BlockSpec((MB, Dout), lambda i: (i, 0)),
      out_shape=jax.ShapeDtypeStruct((NPAD, Dout), jnp.float32),
  )(hmat, degp)

  @functools.partial(
      pl.kernel,
      out_type=jax.ShapeDtypeStruct((NC, NPAD, Dout), jnp.float32),
      mesh=_sc_mesh(),
      scratch_types=[
          pltpu.VMEM((CH * K,), jnp.int32),
          pltpu.VMEM((CH * K,), jnp.int32),
          pltpu.VMEM((K, Dout), jnp.float32),
          pltpu.VMEM((K, Dout), jnp.float32),
          pltpu.VMEM_SHARED((NPAD, Dout), jnp.float32),
          pltpu.SemaphoreType.DMA,
          pltpu.SemaphoreType.DMA,
      ],
  )
  def scat_kernel(g_hbm, ei_hbm, tail_hbm, out_hbm,
                  si_v, di_v, rows_a, rows_b, acc_sp, sem_a, sem_b):
    c = lax.axis_index("c")
    s = lax.axis_index("s")
    wid = c * NS + s

    def zbody(i, carry):
      for jj in range(Dout // L):
        rows_a[i, pl.ds(jj * L, L)] = jnp.zeros((L,), jnp.float32)
      return carry

    lax.fori_loop(0, K, zbody, 0)
    for r in range(TPR // K):
      pltpu.sync_copy(rows_a, acc_sp.at[pl.ds(s * TPR + r * K, K)])
    plsc.subcore_barrier()

    for h in range(H):
      @pl.when(wid < NW - 1)
      def _(h=h):
        pltpu.sync_copy(ei_hbm.at[0, pl.ds(wid * EPW + h * CH * K, CH * K)],
                        si_v)
        pltpu.sync_copy(ei_hbm.at[1, pl.ds(wid * EPW + h * CH * K, CH * K)],
                        di_v)

      @pl.when(wid == NW - 1)
      def _(h=h):
        pltpu.sync_copy(tail_hbm.at[0, pl.ds(h * CH * K, CH * K)], si_v)
        pltpu.sync_copy(tail_hbm.at[1, pl.ds(h * CH * K, CH * K)], di_v)

      pltpu.async_copy(g_hbm.at[si_v.at[pl.ds(0, K)]], rows_a, sem_a)
      pltpu.async_copy(g_hbm.at[si_v.at[pl.ds(K, K)]], rows_b, sem_b)

      def body(jj, carry):
        j = jj * 2
        for rows_v, sem, off in ((rows_a, sem_a, 0), (rows_b, sem_b, 1)):
          pltpu.make_async_copy(g_hbm.at[si_v.at[pl.ds((j + off) * K, K)]],
                                rows_v, sem).wait()
          pltpu.sync_copy(rows_v, acc_sp.at[di_v.at[pl.ds((j + off) * K, K)]],
                          add=True)
          nxt = jnp.minimum((j + off + 2) * K, (CH - 1) * K)
          pltpu.async_copy(g_hbm.at[si_v.at[pl.ds(nxt, K)]], rows_v, sem)
        return carry

      lax.fori_loop(0, CH // 2, body, 0)
      pltpu.make_async_copy(g_hbm.at[si_v.at[pl.ds(0, K)]], rows_a, sem_a).wait()
      pltpu.make_async_copy(g_hbm.at[si_v.at[pl.ds(0, K)]], rows_b, sem_b).wait()
    plsc.subcore_barrier()
    for r in range(TPR // K):
      pltpu.sync_copy(acc_sp.at[pl.ds(s * TPR + r * K, K)],
                      out_hbm.at[c, pl.ds(s * TPR + r * K, K)])

  accp = scat_kernel(g, ei, tail)

  MB2 = 2048

  def fin_body(a0_ref, a1_ref, g_ref, dg_ref, o_ref):
    d = dg_ref[0:1, :] + dg_ref[1:2, :] + 1.0
    dinv = jnp.transpose(lax.rsqrt(jnp.maximum(d, 1.0)), (1, 0))
    tot = a0_ref[0] + a1_ref[0] + g_ref[...]
    o_ref[...] = jnp.maximum(tot * dinv, 0.0)

  out = pl.pallas_call(
      fin_body,
      grid=((N + MB2 - 1) // MB2,),
      in_specs=[
          pl.BlockSpec((1, MB2, Dout), lambda i: (0, i, 0)),
          pl.BlockSpec((1, MB2, Dout), lambda i: (1, i, 0)),
          pl.BlockSpec((MB2, Dout), lambda i: (i, 0)),
          pl.BlockSpec((NC, MB2), lambda i: (0, i)),
      ],
      out_specs=pl.BlockSpec((MB2, Dout), lambda i: (i, 0)),
      out_shape=jax.ShapeDtypeStruct((N, Dout), jnp.float32),
  )(accp, accp, g, degp)

  return out


# speedup vs baseline: 50.8758x; 1.0094x over previous
import functools

import jax
import jax.numpy as jnp
from jax import lax
from jax.experimental import pallas as pl
from jax.experimental.pallas import tpu as pltpu
from jax.experimental.pallas import tpu_sc as plsc

NC = 2
NS = 16
NW = NC * NS
L = 16
K = 128


def _sc_mesh():
  return plsc.VectorSubcoreMesh(core_axis_name="c", subcore_axis_name="s")


def kernel(X, edge_index, W, b):
  N, Din = X.shape
  Dout = W.shape[1]
  E = edge_index.shape[1]

  row_unit = NS * K
  NPAD = ((N + 1 + row_unit - 1) // row_unit) * row_unit
  TPR = NPAD // NS

  C = (((E + NW * K - 1) // (NW * K)) + 3) // 4 * 4
  EPW = C * K
  EPAD = EPW * NW
  H = 2
  CH = C // H

  ei = edge_index.astype(jnp.int32)
  last = (NW - 1) * EPW
  pad = N + jnp.arange(EPAD - E, dtype=jnp.int32) % (NPAD - N)
  padb = jnp.broadcast_to(pad, (2, EPAD - E))
  tail = jnp.concatenate([ei[:, last:], padb], axis=1)

  @functools.partial(
      pl.kernel,
      out_type=jax.ShapeDtypeStruct((NC, NPAD), jnp.float32),
      mesh=_sc_mesh(),
      scratch_types=[
          pltpu.VMEM((EPW,), jnp.int32),
          pltpu.VMEM((K,), jnp.float32),
          pltpu.VMEM((TPR,), jnp.float32),
          pltpu.VMEM_SHARED((NPAD,), jnp.float32),
          pltpu.SemaphoreType.DMA,
      ],
  )
  def deg_kernel(ei_hbm, tail_hbm, out_hbm, idx_v, ones_v, zero_v, deg_sp,
                 sem):
    c = lax.axis_index("c")
    s = lax.axis_index("s")
    wid = c * NS + s
    @pl.when(wid < NW - 1)
    def _():
      pltpu.sync_copy(ei_hbm.at[1, pl.ds(wid * EPW, EPW)], idx_v)

    @pl.when(wid == NW - 1)
    def _():
      pltpu.sync_copy(tail_hbm.at[1], idx_v)

    for i in range(K // L):
      ones_v[pl.ds(i * L, L)] = jnp.ones((L,), jnp.float32)
    for i in range(TPR // L):
      zero_v[pl.ds(i * L, L)] = jnp.zeros((L,), jnp.float32)
    pltpu.sync_copy(zero_v, deg_sp.at[pl.ds(s * TPR, TPR)])
    plsc.subcore_barrier()

    def body(j, carry):
      pltpu.async_copy(ones_v, deg_sp.at[idx_v.at[pl.ds(j * K, K)]], sem,
                       add=True)
      return carry

    lax.fori_loop(0, C, body, 0)
    pltpu.make_async_copy(tail_hbm.at[1], idx_v, sem).wait()
    plsc.subcore_barrier()
    pltpu.sync_copy(deg_sp.at[pl.ds(s * TPR, TPR)],
                    out_hbm.at[c, pl.ds(s * TPR, TPR)])

  degp = deg_kernel(ei, tail)

  MB = 2048

  def mm_body(x_ref, w_ref, b_ref, dg_ref, g_ref):
    d = dg_ref[0:1, :] + dg_ref[1:2, :] + 1.0
    dinv = jnp.transpose(lax.rsqrt(jnp.maximum(d, 1.0)), (1, 0))
    h = jnp.dot(x_ref[...], w_ref[...],
                preferred_element_type=jnp.float32) + b_ref[...]
    g_ref[...] = h * dinv

  g = pl.pallas_call(
      mm_body,
      grid=((N + MB - 1) // MB,),
      in_specs=[
          pl.BlockSpec((MB, Din), lambda i: (i, 0)),
          pl.BlockSpec((Din, Dout), lambda i: (0, 0)),
          pl.BlockSpec((1, Dout), lambda i: (0, 0)),
          pl.BlockSpec((NC, MB), lambda i: (0, i)),
      ],
      out_specs=pl.BlockSpec((MB, Dout), lambda i: (i, 0)),
      out_shape=jax.ShapeDtypeStruct((NPAD, Dout), jnp.float32),
  )(X, W, b.reshape(1, Dout), degp)

  @functools.partial(
      pl.kernel,
      out_type=jax.ShapeDtypeStruct((NC, NPAD, Dout), jnp.float32),
      mesh=_sc_mesh(),
      scratch_types=[
          pltpu.VMEM((CH * K,), jnp.int32),
          pltpu.VMEM((CH * K,), jnp.int32),
          pltpu.VMEM((K, Dout), jnp.float32),
          pltpu.VMEM((K, Dout), jnp.float32),
          pltpu.VMEM_SHARED((NPAD, Dout), jnp.float32),
          pltpu.SemaphoreType.DMA,
          pltpu.SemaphoreType.DMA,
      ],
  )
  def scat_kernel(g_hbm, ei_hbm, tail_hbm, out_hbm,
                  si_v, di_v, rows_a, rows_b, acc_sp, sem_a, sem_b):
    c = lax.axis_index("c")
    s = lax.axis_index("s")
    wid = c * NS + s

    def zbody(i, carry):
      for jj in range(Dout // L):
        rows_a[i, pl.ds(jj * L, L)] = jnp.zeros((L,), jnp.float32)
      return carry

    lax.fori_loop(0, K, zbody, 0)
    for r in range(TPR // K):
      pltpu.sync_copy(rows_a, acc_sp.at[pl.ds(s * TPR + r * K, K)])
    plsc.subcore_barrier()

    for h in range(H):
      @pl.when(wid < NW - 1)
      def _(h=h):
        pltpu.sync_copy(ei_hbm.at[0, pl.ds(wid * EPW + h * CH * K, CH * K)],
                        si_v)
        pltpu.sync_copy(ei_hbm.at[1, pl.ds(wid * EPW + h * CH * K, CH * K)],
                        di_v)

      @pl.when(wid == NW - 1)
      def _(h=h):
        pltpu.sync_copy(tail_hbm.at[0, pl.ds(h * CH * K, CH * K)], si_v)
        pltpu.sync_copy(tail_hbm.at[1, pl.ds(h * CH * K, CH * K)], di_v)

      pltpu.async_copy(g_hbm.at[si_v.at[pl.ds(0, K)]], rows_a, sem_a)
      pltpu.async_copy(g_hbm.at[si_v.at[pl.ds(K, K)]], rows_b, sem_b)

      def body(jj, carry):
        j = jj * 2
        for rows_v, sem, off in ((rows_a, sem_a, 0), (rows_b, sem_b, 1)):
          pltpu.make_async_copy(g_hbm.at[si_v.at[pl.ds((j + off) * K, K)]],
                                rows_v, sem).wait()
          pltpu.sync_copy(rows_v, acc_sp.at[di_v.at[pl.ds((j + off) * K, K)]],
                          add=True)
          nxt = jnp.minimum((j + off + 2) * K, (CH - 1) * K)
          pltpu.async_copy(g_hbm.at[si_v.at[pl.ds(nxt, K)]], rows_v, sem)
        return carry

      lax.fori_loop(0, CH // 2, body, 0)
      pltpu.make_async_copy(g_hbm.at[si_v.at[pl.ds(0, K)]], rows_a, sem_a).wait()
      pltpu.make_async_copy(g_hbm.at[si_v.at[pl.ds(0, K)]], rows_b, sem_b).wait()
    plsc.subcore_barrier()
    for r in range(TPR // K):
      pltpu.sync_copy(acc_sp.at[pl.ds(s * TPR + r * K, K)],
                      out_hbm.at[c, pl.ds(s * TPR + r * K, K)])

  accp = scat_kernel(g, ei, tail)

  MB2 = 2048

  def fin_body(a0_ref, a1_ref, g_ref, dg_ref, o_ref):
    d = dg_ref[0:1, :] + dg_ref[1:2, :] + 1.0
    dinv = jnp.transpose(lax.rsqrt(jnp.maximum(d, 1.0)), (1, 0))
    tot = a0_ref[0] + a1_ref[0] + g_ref[...]
    o_ref[...] = jnp.maximum(tot * dinv, 0.0)

  out = pl.pallas_call(
      fin_body,
      grid=((N + MB2 - 1) // MB2,),
      in_specs=[
          pl.BlockSpec((1, MB2, Dout), lambda i: (0, i, 0)),
          pl.BlockSpec((1, MB2, Dout), lambda i: (1, i, 0)),
          pl.BlockSpec((MB2, Dout), lambda i: (i, 0)),
          pl.BlockSpec((NC, MB2), lambda i: (0, i)),
      ],
      out_specs=pl.BlockSpec((MB2, Dout), lambda i: (i, 0)),
      out_shape=jax.ShapeDtypeStruct((N, Dout), jnp.float32),
  )(accp, accp, g, degp)

  return out
